# Initial kernel scaffold; baseline (speedup 1.0000x reference)
import jax
import jax.numpy as jnp
from jax.experimental import pallas as pl


def kernel(x, emb):
    raise NotImplementedError("write your pallas kernel here")


# speedup vs baseline: 1.6442x; 1.6442x over previous
import functools

import jax
import jax.numpy as jnp
from jax import lax
from jax.experimental import pallas as pl
from jax.experimental.pallas import tpu as pltpu
from jax.experimental.pallas import tpu_sc as plsc


def _make_kernel(batch, seq_len, dim, dtype):
    info = plsc.get_sparse_core_info()
    nc, ns = info.num_cores, info.num_subcores
    nw = nc * ns
    assert seq_len % nw == 0
    rows_per_w = seq_len // nw
    chunk = min(rows_per_w, 64)
    assert rows_per_w % chunk == 0
    n_chunks = rows_per_w // chunk

    mesh = plsc.VectorSubcoreMesh(core_axis_name="c", subcore_axis_name="s")

    @functools.partial(
        pl.kernel,
        mesh=mesh,
        out_type=jax.ShapeDtypeStruct((batch, seq_len, dim), dtype),
        scratch_types=[
            pltpu.VMEM((chunk, dim), dtype),
            pltpu.SemaphoreType.DMA,
            pltpu.SemaphoreType.DMA,
        ],
    )
    def emb_broadcast(emb_hbm, out_hbm, buf, read_sem, write_sem):
        wid = lax.axis_index("s") * nc + lax.axis_index("c")
        base = wid * rows_per_w

        def body(i, carry):
            row0 = base + i * chunk
            pltpu.async_copy(emb_hbm.at[pl.ds(row0, chunk)], buf, read_sem).wait()
            copies = [
                pltpu.async_copy(buf, out_hbm.at[b, pl.ds(row0, chunk)], write_sem)
                for b in range(batch)
            ]
            for c in copies:
                c.wait()
            return carry

        lax.fori_loop(0, n_chunks, body, 0)

    return emb_broadcast


def kernel(x, emb):
    batch, seq_len, _ = x.shape
    f = _make_kernel(batch, seq_len, emb.shape[1], emb.dtype)
    return f(emb)
